# baseline (speedup 1.0000x reference)
import functools

import jax
import jax.numpy as jnp
from jax import lax
from jax.experimental import pallas as pl
from jax.experimental.pallas import tpu as pltpu
from jax.experimental.pallas import tpu_sc as plsc

NUM_EMB = 1000000
D = 64
DP = 128
B_TOK = 4096
T_TOK = 200
TP = 256
NC = 2
NS = 16
NW = NC * NS
BPW = B_TOK // NW
CHUNK = 128
REM = T_TOK - CHUNK


def _sc_gather(table, idx3):
    mesh = plsc.VectorSubcoreMesh(core_axis_name="c", subcore_axis_name="s")

    @functools.partial(
        pl.kernel,
        mesh=mesh,
        out_type=jax.ShapeDtypeStruct((B_TOK, T_TOK, DP), jnp.float32),
        compiler_params=pltpu.CompilerParams(use_tc_tiling_on_sc=True),
        scratch_types=[
            pltpu.VMEM((BPW, TP), jnp.int32),
            pltpu.VMEM((2 * T_TOK, DP), jnp.float32),
            pltpu.SemaphoreType.DMA,
            pltpu.SemaphoreType.DMA,
        ],
    )
    def k(table_hbm, idx_hbm, out_hbm, idx_v, rows_v, sem_in, sem_out):
        wid = lax.axis_index("s") * NC + lax.axis_index("c")
        pltpu.sync_copy(idx_hbm.at[wid], idx_v)

        def gathers(j, half):
            return (
                pltpu.make_async_copy(
                    table_hbm.at[idx_v.at[j, pl.ds(0, CHUNK)]],
                    rows_v.at[pl.ds(half * T_TOK, CHUNK)],
                    sem_in,
                ),
                pltpu.make_async_copy(
                    table_hbm.at[idx_v.at[j, pl.ds(CHUNK, REM)]],
                    rows_v.at[pl.ds(half * T_TOK + CHUNK, REM)],
                    sem_in,
                ),
            )

        def out_copy(j, half):
            return pltpu.make_async_copy(
                rows_v.at[pl.ds(half * T_TOK, T_TOK)],
                out_hbm.at[wid * BPW + j],
                sem_out,
            )

        for c in gathers(0, 0):
            c.start()

        def body(j, carry):
            half = lax.rem(j, 2)

            @pl.when(j >= 1)
            def _():
                out_copy(j - 1, 1 - half).wait()

            @pl.when(j + 1 < BPW)
            def _():
                for c in gathers(j + 1, 1 - half):
                    c.start()

            for c in gathers(j, half):
                c.wait()
            out_copy(j, half).start()
            return carry

        lax.fori_loop(0, BPW, body, 0)
        out_copy(BPW - 1, (BPW - 1) % 2).wait()

    return k(table, idx3)


def kernel(token_ids, embeddings):
    pad_proj = jnp.concatenate(
        [jnp.eye(D, dtype=jnp.float32),
         jnp.zeros((D, DP - D), jnp.float32)], axis=1)
    table = jax.lax.dot(embeddings, pad_proj,
                        precision=jax.lax.Precision.HIGHEST)
    idx3 = jnp.pad(token_ids.reshape(NW, BPW, T_TOK), ((0, 0), (0, 0), (0, TP - T_TOK)))
    out = _sc_gather(table, idx3)
    return out[:, :, :D]


# speedup vs baseline: 1.2740x; 1.2740x over previous
import functools

import jax
import jax.numpy as jnp
from jax import lax
from jax.experimental import pallas as pl
from jax.experimental.pallas import tpu as pltpu
from jax.experimental.pallas import tpu_sc as plsc

NUM_EMB = 1000000
D = 64
DP = 128
B_TOK = 4096
T_TOK = 200
TP = 256
NC = 2
NS = 16
NW = NC * NS
BPW = B_TOK // NW
CHUNK = 128
REM = T_TOK - CHUNK


def _sc_gather(table, idx3):
    mesh = plsc.VectorSubcoreMesh(core_axis_name="c", subcore_axis_name="s")

    @functools.partial(
        pl.kernel,
        mesh=mesh,
        out_type=jax.ShapeDtypeStruct((B_TOK, T_TOK, DP), jnp.float32),
        compiler_params=pltpu.CompilerParams(use_tc_tiling_on_sc=True),
        scratch_types=[
            pltpu.VMEM((BPW, TP), jnp.int32),
            pltpu.VMEM((2 * T_TOK, DP), jnp.float32),
            pltpu.SemaphoreType.DMA,
            pltpu.SemaphoreType.DMA,
        ],
    )
    def k(table_hbm, idx_hbm, out_hbm, idx_v, rows_v, sem_in, sem_out):
        wid = lax.axis_index("s") * NC + lax.axis_index("c")
        pltpu.sync_copy(idx_hbm.at[wid], idx_v)

        def gathers(j, half):
            return (
                pltpu.make_async_copy(
                    table_hbm.at[idx_v.at[j, pl.ds(0, CHUNK)]],
                    rows_v.at[pl.ds(half * T_TOK, CHUNK)],
                    sem_in,
                ),
                pltpu.make_async_copy(
                    table_hbm.at[idx_v.at[j, pl.ds(CHUNK, REM)]],
                    rows_v.at[pl.ds(half * T_TOK + CHUNK, REM)],
                    sem_in,
                ),
            )

        def out_copy(j, half):
            return pltpu.make_async_copy(
                rows_v.at[pl.ds(half * T_TOK, T_TOK)],
                out_hbm.at[wid * BPW + j],
                sem_out,
            )

        for c in gathers(0, 0):
            c.start()

        def body(j, carry):
            half = lax.rem(j, 2)

            @pl.when(j >= 1)
            def _():
                out_copy(j - 1, 1 - half).wait()

            @pl.when(j + 1 < BPW)
            def _():
                for c in gathers(j + 1, 1 - half):
                    c.start()

            for c in gathers(j, half):
                c.wait()
            out_copy(j, half).start()
            return carry

        lax.fori_loop(0, BPW, body, 0)
        out_copy(BPW - 1, (BPW - 1) % 2).wait()

    return k(table, idx3)


def kernel(token_ids, embeddings):
    table = jnp.pad(embeddings.reshape(NUM_EMB // 8, 8, D),
                    ((0, 0), (0, 0), (0, DP - D))).reshape(NUM_EMB, DP)
    idx3 = jnp.pad(token_ids.reshape(NW, BPW, T_TOK), ((0, 0), (0, 0), (0, TP - T_TOK)))
    out = _sc_gather(table, idx3)
    return out[:, :, :D]


# speedup vs baseline: 1.2793x; 1.0042x over previous
import functools

import jax
import jax.numpy as jnp
from jax import lax
from jax.experimental import pallas as pl
from jax.experimental.pallas import tpu as pltpu
from jax.experimental.pallas import tpu_sc as plsc

NUM_EMB = 1000000
D = 64
DP = 128
B_TOK = 4096
T_TOK = 200
TP = 256
NC = 2
NS = 16
NW = NC * NS
BPW = B_TOK // NW
CHUNK = 128
REM = T_TOK - CHUNK


def _sc_gather(table, idx3):
    mesh = plsc.VectorSubcoreMesh(core_axis_name="c", subcore_axis_name="s")

    @functools.partial(
        pl.kernel,
        mesh=mesh,
        out_type=jax.ShapeDtypeStruct((B_TOK, T_TOK, DP), jnp.float32),
        compiler_params=pltpu.CompilerParams(use_tc_tiling_on_sc=True),
        scratch_types=[
            pltpu.VMEM((BPW, TP), jnp.int32),
            pltpu.VMEM((3 * T_TOK, DP), jnp.float32),
            pltpu.SemaphoreType.DMA,
            pltpu.SemaphoreType.DMA,
        ],
    )
    def k(table_hbm, idx_hbm, out_hbm, idx_v, rows_v, sem_in, sem_out):
        wid = lax.axis_index("s") * NC + lax.axis_index("c")
        pltpu.sync_copy(idx_hbm.at[wid], idx_v)

        def gathers(j, slot):
            return (
                pltpu.make_async_copy(
                    table_hbm.at[idx_v.at[j, pl.ds(0, CHUNK)]],
                    rows_v.at[pl.ds(slot * T_TOK, CHUNK)],
                    sem_in,
                ),
                pltpu.make_async_copy(
                    table_hbm.at[idx_v.at[j, pl.ds(CHUNK, REM)]],
                    rows_v.at[pl.ds(slot * T_TOK + CHUNK, REM)],
                    sem_in,
                ),
            )

        def out_copy(j, slot):
            return pltpu.make_async_copy(
                rows_v.at[pl.ds(slot * T_TOK, T_TOK)],
                out_hbm.at[wid * BPW + j],
                sem_out,
            )

        for c in gathers(0, 0):
            c.start()

        def body(j, carry):
            slot = lax.rem(j, 3)

            @pl.when(j >= 2)
            def _():
                out_copy(j - 2, lax.rem(j + 1, 3)).wait()

            @pl.when(j + 1 < BPW)
            def _():
                for c in gathers(j + 1, lax.rem(j + 1, 3)):
                    c.start()

            for c in gathers(j, slot):
                c.wait()
            out_copy(j, slot).start()
            return carry

        lax.fori_loop(0, BPW, body, 0)
        out_copy(BPW - 2, (BPW - 2) % 3).wait()
        out_copy(BPW - 1, (BPW - 1) % 3).wait()

    return k(table, idx3)


def kernel(token_ids, embeddings):
    table = jnp.pad(embeddings.reshape(NUM_EMB // 8, 8, D),
                    ((0, 0), (0, 0), (0, DP - D))).reshape(NUM_EMB, DP)
    idx3 = jnp.pad(token_ids.reshape(NW, BPW, T_TOK), ((0, 0), (0, 0), (0, TP - T_TOK)))
    out = _sc_gather(table, idx3)
    return out[:, :, :D]
